# baseline (speedup 1.0000x reference)
import jax
import jax.numpy as jnp
from jax import lax
from jax.experimental import pallas as pl
from jax.experimental.pallas import tpu as pltpu
from jax.experimental.pallas import tpu_sc as plsc

N = 10000
E = 320000
DF = 128
HD = 64
H = 8
OC = 8
NT = 80
NP = 10240
NC, NS, L = 2, 16, 16
TILES = NC * NS
IGRP = 8
GEDGE = IGRP * 128
NGRP = 10
EPT = GEDGE * NGRP
EP = EPT * TILES
EROWS = EP // 128
RPT = EPT // 128
SUBC = 256
NBUF = 4
RPN = NP // NS

_MESH = plsc.VectorSubcoreMesh(core_axis_name="c", subcore_axis_name="s",
                               num_cores=NC, num_subcores=NS)
_SC_PARAMS = pltpu.CompilerParams(needs_layout_passes=False,
                                  use_tc_tiling_on_sc=False)


def _t1_body(x_ref, w1_ref, am_ref, nt_ref, ad_ref):
    h = jnp.dot(x_ref[...], w1_ref[...], preferred_element_type=jnp.float32)
    asd = jnp.dot(h, am_ref[...], preferred_element_type=jnp.float32)
    nt_ref[...] = jnp.concatenate([h, asd], axis=1)
    ad_ref[...] = asd


_t1 = pl.pallas_call(
    _t1_body,
    out_shape=[jax.ShapeDtypeStruct((NP, NT), jnp.float32),
               jax.ShapeDtypeStruct((NP, 2 * H), jnp.float32)],
)


def _s1_body(srcr, dstr, nt_hbm, adr_hbm, z128,
             outp,
             sidx, didx, rows, adrows, acc, gsem, ssem):
    c = lax.axis_index("c")
    s = lax.axis_index("s")
    wid = c * NS + s

    pltpu.sync_copy(z128.at[pl.ds(s * RPN, RPN)], acc.at[pl.ds(s * RPN, RPN)])
    pltpu.sync_copy(srcr.at[pl.ds(wid * RPT, RPT)], sidx)
    pltpu.sync_copy(dstr.at[pl.ds(wid * RPT, RPT)], didx)
    plsc.subcore_barrier()

    lane = lax.iota(jnp.int32, L)
    col8 = lane >> 3
    cola = lane & 7
    colw = cola + HD

    def start_gather(tt, b):
        pltpu.async_copy(nt_hbm.at[sidx.at[tt]], rows.at[b], gsem.at[b])
        pltpu.async_copy(adr_hbm.at[didx.at[tt]], adrows.at[b], gsem.at[b])

    def wait_gather(b):
        pltpu.make_async_copy(nt_hbm.at[sidx.at[0]], rows.at[b],
                              gsem.at[b]).wait()
        pltpu.make_async_copy(adr_hbm.at[didx.at[0]], adrows.at[b],
                              gsem.at[b]).wait()

    def wait_scatter(b):
        pltpu.make_async_copy(rows.at[b], acc.at[didx.at[0]],
                              ssem.at[b]).wait()

    for pb in range(NBUF - 1):
        start_gather(pb, pb)

    def chunk_body(t, carry):
        b = t & (NBUF - 1)

        @pl.when(t < RPT - (NBUF - 1))
        def _():
            start_gather(t + NBUF - 1, (t + NBUF - 1) & (NBUF - 1))

        wait_gather(b)

        @plsc.parallel_loop(0, 64, unroll=4)
        def pair_body(p):
            e = p * 2
            ridx = e + col8
            adp = plsc.load_gather(adrows.at[b], [ridx, cola + H])
            asp = plsc.load_gather(rows.at[b], [ridx, colw])
            al = asp + adp
            ea = jnp.exp(jnp.maximum(al, 0.2 * al))
            plsc.store_scatter(rows.at[b], [ridx, colw], ea)
            for q in range(2):
                ee = e + q
                eidx = jnp.full((L,), ee, jnp.int32)
                for r in range(4):
                    w = plsc.load_gather(rows.at[b],
                                         [eidx, col8 + (HD + 2 * r)])
                    rows[b, ee, pl.ds(r * L, L)] = (
                        rows[b, ee, pl.ds(r * L, L)] * w)

        pltpu.async_copy(rows.at[b], acc.at[didx.at[t]], ssem.at[b],
                         add=True)
        wait_scatter(b)
        return carry

    lax.fori_loop(0, RPT, chunk_body, 0)
    plsc.subcore_barrier()
    pltpu.sync_copy(acc.at[pl.ds(s * RPN, RPN)],
                    outp.at[c, pl.ds(s * RPN, RPN)])


_s1 = pl.kernel(
    _s1_body,
    out_type=jax.ShapeDtypeStruct((NC, NP, NT), jnp.float32),
    mesh=_MESH,
    compiler_params=_SC_PARAMS,
    scratch_types=[
        pltpu.VMEM((RPT, 128), jnp.int32),
        pltpu.VMEM((RPT, 128), jnp.int32),
        pltpu.VMEM((NBUF, 128, NT), jnp.float32),
        pltpu.VMEM((NBUF, 128, 2 * H), jnp.float32),
        pltpu.VMEM_SHARED((NP, NT), jnp.float32),
        pltpu.SemaphoreType.DMA((NBUF,)),
        pltpu.SemaphoreType.DMA((NBUF,)),
    ],
)


T2B = 2048


def _t2_body(op_ref, nt_ref, b1_ref, w2_ref, h2_ref):
    i = pl.program_id(0)
    nt = nt_ref[...]
    h1 = nt[:, :HD]
    sa = nt[:, HD:HD + H] + nt[:, HD + H:HD + 2 * H]
    se = jnp.exp(jnp.maximum(sa, 0.2 * sa))
    se64 = jnp.broadcast_to(se[:, :, None], (T2B, H, OC)).reshape(T2B, HD)
    outu = op_ref[0][:, :HD] + op_ref[1][:, :HD] + h1 * se64
    den = op_ref[0][:, HD:HD + H] + op_ref[1][:, HD:HD + H] + se
    den64 = jnp.broadcast_to((den + 1e-16)[:, :, None],
                             (T2B, H, OC)).reshape(T2B, HD)
    hrel = jnp.maximum(outu / den64 + b1_ref[...][None, :], 0.0)
    row = i * T2B + lax.broadcasted_iota(jnp.int32, (T2B, 1), 0)
    hrel = jnp.where(row < N, hrel, 0.0)
    h2_ref[...] = jnp.dot(hrel, w2_ref[...], preferred_element_type=jnp.float32)


_t2 = pl.pallas_call(
    _t2_body,
    grid=(NP // T2B,),
    in_specs=[
        pl.BlockSpec((NC, T2B, NT), lambda i: (0, i, 0)),
        pl.BlockSpec((T2B, NT), lambda i: (i, 0)),
        pl.BlockSpec((HD,), lambda i: (0,)),
        pl.BlockSpec((HD, 1), lambda i: (0, 0)),
    ],
    out_specs=pl.BlockSpec((T2B, 1), lambda i: (i, 0)),
    out_shape=jax.ShapeDtypeStruct((NP, 1), jnp.float32),
)


def _s2_body(srcr, dstr, h2_hbm, z2, a2_hbm,
             out2,
             sidx, didx, h2t, val, a2v, acc2, sem):
    c = lax.axis_index("c")
    s = lax.axis_index("s")
    wid = c * NS + s

    pltpu.sync_copy(z2.at[pl.ds(s * RPN, RPN)], acc2.at[pl.ds(s * RPN, RPN)])
    pltpu.sync_copy(z2.at[pl.ds(0, GEDGE)], val)
    pltpu.sync_copy(h2_hbm, h2t)
    pltpu.sync_copy(a2_hbm, a2v)
    pltpu.sync_copy(srcr.at[pl.ds(wid * RPT, RPT)], sidx)
    pltpu.sync_copy(dstr.at[pl.ds(wid * RPT, RPT)], didx)
    plsc.subcore_barrier()

    lane = lax.iota(jnp.int32, L)
    zero16 = jnp.zeros((L,), jnp.int32)
    one16 = zero16 + 1
    a2s = a2v[0, :]
    a2d = a2v[1, :]

    def group_body(g, carry):
        for j in range(IGRP):
            for k in range(128 // L):
                sv = sidx[g * IGRP + j, pl.ds(k * L, L)]
                dv = didx[g * IGRP + j, pl.ds(k * L, L)]
                hs = plsc.load_gather(h2t, [sv])
                hd = plsc.load_gather(h2t, [dv])
                al = a2s * hs + a2d * hd
                ea = jnp.exp(jnp.maximum(al, 0.2 * al))
                gg = j * 128 + k * L + lane
                plsc.store_scatter(val, [gg, zero16], ea * hs)
                plsc.store_scatter(val, [gg, one16], ea)
        for j in range(IGRP):
            pltpu.sync_copy(val.at[pl.ds(j * 128, 128)],
                            acc2.at[didx.at[g * IGRP + j]], add=True)
        return carry

    lax.fori_loop(0, NGRP, group_body, 0)
    plsc.subcore_barrier()
    pltpu.sync_copy(acc2.at[pl.ds(s * RPN, RPN)],
                    out2.at[c, pl.ds(s * RPN, RPN)])


_s2 = pl.kernel(
    _s2_body,
    out_type=jax.ShapeDtypeStruct((NC, NP, 2 * H), jnp.float32),
    mesh=_MESH,
    compiler_params=_SC_PARAMS,
    scratch_types=[
        pltpu.VMEM((RPT, 128), jnp.int32),
        pltpu.VMEM((RPT, 128), jnp.int32),
        pltpu.VMEM((NP,), jnp.float32),
        pltpu.VMEM((GEDGE, 2 * H), jnp.float32),
        pltpu.VMEM((2, L), jnp.float32),
        pltpu.VMEM_SHARED((NP, 2 * H), jnp.float32),
        pltpu.SemaphoreType.DMA,
    ],
)


def _t3_body(o2_ref, h2_ref, sc_ref, out_ref):
    a2s = sc_ref[0, 0]
    a2d = sc_ref[0, 1]
    b2v = sc_ref[0, 2]
    h2 = h2_ref[...]
    sa = (a2s + a2d) * h2
    sea = jnp.exp(jnp.maximum(sa, 0.2 * sa))
    num = o2_ref[0][:, 0:1] + o2_ref[1][:, 0:1] + sea * h2
    den = o2_ref[0][:, 1:2] + o2_ref[1][:, 1:2] + sea
    node = num / (den + 1e-16)
    valid = lax.broadcasted_iota(jnp.int32, (NP, 1), 0) < N
    node = jnp.where(valid, node, 0.0)
    out_ref[...] = (jnp.sum(node) / N + b2v).reshape(1, 1)


_t3 = pl.pallas_call(
    _t3_body,
    out_shape=jax.ShapeDtypeStruct((1, 1), jnp.float32),
)


def kernel(x, edge_index, W1, a_src1, a_dst1, b1, W2, a_src2, a_dst2, b2):
    x = x.astype(jnp.float32)
    src = edge_index[0].astype(jnp.int32)
    dst = edge_index[1].astype(jnp.int32)
    srcr = jnp.pad(src, (0, EP - E), constant_values=N).reshape(EROWS, 128)
    dstr = jnp.pad(dst, (0, EP - E), constant_values=N).reshape(EROWS, 128)
    xp = jnp.pad(x, ((0, NP - N), (0, 0)))

    eye = jnp.eye(H, dtype=jnp.float32)
    a_mat = jnp.concatenate(
        [(eye[:, None, :] * a_src1[:, :, None]).reshape(HD, H),
         (eye[:, None, :] * a_dst1[:, :, None]).reshape(HD, H)], axis=1)

    ntab, adr = _t1(xp, W1, a_mat)

    z128 = jnp.zeros((NP, NT), jnp.float32)
    outp = _s1(srcr, dstr, ntab, adr, z128)

    h2 = _t2(outp, ntab, b1, W2)

    a2v = jnp.stack([jnp.full((L,), a_src2[0, 0], jnp.float32),
                     jnp.full((L,), a_dst2[0, 0], jnp.float32)])
    z2 = jnp.zeros((NP, 2 * H), jnp.float32)
    out2 = _s2(srcr, dstr, h2[:, 0], z2, a2v)

    sc3 = jnp.stack([a_src2[0, 0], a_dst2[0, 0], b2[0],
                     jnp.float32(0.0)]).reshape(1, 4)
    return _t3(out2, h2, sc3)


# speedup vs baseline: 1.0199x; 1.0199x over previous
import jax
import jax.numpy as jnp
from jax import lax
from jax.experimental import pallas as pl
from jax.experimental.pallas import tpu as pltpu
from jax.experimental.pallas import tpu_sc as plsc

N = 10000
E = 320000
DF = 128
HD = 64
H = 8
OC = 8
NT = 80
NP = 10240
NC, NS, L = 2, 16, 16
TILES = NC * NS
IGRP = 8
GEDGE = IGRP * 128
NGRP = 10
EPT = GEDGE * NGRP
EP = EPT * TILES
EROWS = EP // 128
RPT = EPT // 128
SUBC = 256
NBUF = 4
RPN = NP // NS

_MESH = plsc.VectorSubcoreMesh(core_axis_name="c", subcore_axis_name="s",
                               num_cores=NC, num_subcores=NS)
_SC_PARAMS = pltpu.CompilerParams(needs_layout_passes=False,
                                  use_tc_tiling_on_sc=False)


def _t1_body(x_ref, w1_ref, am_ref, nt_ref, ad_ref):
    h = jnp.dot(x_ref[...], w1_ref[...], preferred_element_type=jnp.float32)
    asd = jnp.dot(h, am_ref[...], preferred_element_type=jnp.float32)
    nt_ref[...] = jnp.concatenate([h, asd], axis=1)
    ad_ref[...] = asd


_t1 = pl.pallas_call(
    _t1_body,
    out_shape=[jax.ShapeDtypeStruct((NP, NT), jnp.float32),
               jax.ShapeDtypeStruct((NP, 2 * H), jnp.float32)],
)


def _s1_body(srcr, dstr, nt_hbm, adr_hbm, z128,
             outp,
             sidx, didx, rows, adrows, acc, gsem, ssem):
    c = lax.axis_index("c")
    s = lax.axis_index("s")
    wid = c * NS + s

    pltpu.sync_copy(z128.at[pl.ds(s * RPN, RPN)], acc.at[pl.ds(s * RPN, RPN)])
    pltpu.sync_copy(srcr.at[pl.ds(wid * RPT, RPT)], sidx)
    pltpu.sync_copy(dstr.at[pl.ds(wid * RPT, RPT)], didx)
    plsc.subcore_barrier()

    lane = lax.iota(jnp.int32, L)
    col8 = lane >> 3
    cola = lane & 7
    colw = cola + HD

    def start_gather(tt, b):
        pltpu.async_copy(nt_hbm.at[sidx.at[tt]], rows.at[b], gsem.at[b])
        pltpu.async_copy(adr_hbm.at[didx.at[tt]], adrows.at[b], gsem.at[b])

    def wait_gather(b):
        pltpu.make_async_copy(nt_hbm.at[sidx.at[0]], rows.at[b],
                              gsem.at[b]).wait()
        pltpu.make_async_copy(adr_hbm.at[didx.at[0]], adrows.at[b],
                              gsem.at[b]).wait()

    def wait_scatter(b):
        pltpu.make_async_copy(rows.at[b], acc.at[didx.at[0]],
                              ssem.at[b]).wait()

    for pb in range(NBUF - 1):
        start_gather(pb, pb)

    def chunk_body(t, carry):
        b = t & (NBUF - 1)

        @pl.when(t < RPT - (NBUF - 1))
        def _():
            start_gather(t + NBUF - 1, (t + NBUF - 1) & (NBUF - 1))

        wait_gather(b)

        @plsc.parallel_loop(0, 64, unroll=4)
        def pair_body(p):
            e = p * 2
            ridx = e + col8
            adp = plsc.load_gather(adrows.at[b], [ridx, cola + H])
            asp = plsc.load_gather(rows.at[b], [ridx, colw])
            al = asp + adp
            ea = jnp.exp(jnp.maximum(al, 0.2 * al))
            plsc.store_scatter(rows.at[b], [ridx, colw], ea)
            for q in range(2):
                ee = e + q
                eidx = jnp.full((L,), ee, jnp.int32)
                for r in range(4):
                    w = plsc.load_gather(rows.at[b],
                                         [eidx, col8 + (HD + 2 * r)])
                    rows[b, ee, pl.ds(r * L, L)] = (
                        rows[b, ee, pl.ds(r * L, L)] * w)

        pltpu.async_copy(rows.at[b], acc.at[didx.at[t]], ssem.at[b],
                         add=True)
        wait_scatter(b)
        return carry

    lax.fori_loop(0, RPT, chunk_body, 0)
    plsc.subcore_barrier()
    pltpu.sync_copy(acc.at[pl.ds(s * RPN, RPN)],
                    outp.at[c, pl.ds(s * RPN, RPN)])


_s1 = pl.kernel(
    _s1_body,
    out_type=jax.ShapeDtypeStruct((NC, NP, NT), jnp.float32),
    mesh=_MESH,
    compiler_params=_SC_PARAMS,
    scratch_types=[
        pltpu.VMEM((RPT, 128), jnp.int32),
        pltpu.VMEM((RPT, 128), jnp.int32),
        pltpu.VMEM((NBUF, 128, NT), jnp.float32),
        pltpu.VMEM((NBUF, 128, 2 * H), jnp.float32),
        pltpu.VMEM_SHARED((NP, NT), jnp.float32),
        pltpu.SemaphoreType.DMA((NBUF,)),
        pltpu.SemaphoreType.DMA((NBUF,)),
    ],
)


T2B = 2048


def _t2_body(op_ref, nt_ref, b1_ref, w2_ref, h2_ref):
    i = pl.program_id(0)
    nt = nt_ref[...]
    h1 = nt[:, :HD]
    sa = nt[:, HD:HD + H] + nt[:, HD + H:HD + 2 * H]
    se = jnp.exp(jnp.maximum(sa, 0.2 * sa))
    se64 = jnp.broadcast_to(se[:, :, None], (T2B, H, OC)).reshape(T2B, HD)
    outu = op_ref[0][:, :HD] + op_ref[1][:, :HD] + h1 * se64
    den = op_ref[0][:, HD:HD + H] + op_ref[1][:, HD:HD + H] + se
    den64 = jnp.broadcast_to((den + 1e-16)[:, :, None],
                             (T2B, H, OC)).reshape(T2B, HD)
    hrel = jnp.maximum(outu / den64 + b1_ref[...][None, :], 0.0)
    row = i * T2B + lax.broadcasted_iota(jnp.int32, (T2B, 1), 0)
    hrel = jnp.where(row < N, hrel, 0.0)
    h2_ref[...] = jnp.dot(hrel, w2_ref[...], preferred_element_type=jnp.float32)


_t2 = pl.pallas_call(
    _t2_body,
    grid=(NP // T2B,),
    in_specs=[
        pl.BlockSpec((NC, T2B, NT), lambda i: (0, i, 0)),
        pl.BlockSpec((T2B, NT), lambda i: (i, 0)),
        pl.BlockSpec((HD,), lambda i: (0,)),
        pl.BlockSpec((HD, 1), lambda i: (0, 0)),
    ],
    out_specs=pl.BlockSpec((T2B, 1), lambda i: (i, 0)),
    out_shape=jax.ShapeDtypeStruct((NP, 1), jnp.float32),
)


def _s2_body(srcr, dstr, h2_hbm, z2, a2_hbm,
             out2,
             sidx, didx, h2t, val, a2v, acc2, sem):
    c = lax.axis_index("c")
    s = lax.axis_index("s")
    wid = c * NS + s

    pltpu.sync_copy(z2.at[pl.ds(s * RPN, RPN)], acc2.at[pl.ds(s * RPN, RPN)])
    pltpu.sync_copy(z2.at[pl.ds(0, GEDGE)], val)
    pltpu.sync_copy(h2_hbm, h2t)
    pltpu.sync_copy(a2_hbm, a2v)
    pltpu.sync_copy(srcr.at[pl.ds(wid * RPT, RPT)], sidx)
    pltpu.sync_copy(dstr.at[pl.ds(wid * RPT, RPT)], didx)
    plsc.subcore_barrier()

    lane = lax.iota(jnp.int32, L)
    zero16 = jnp.zeros((L,), jnp.int32)
    one16 = zero16 + 1
    a2s = a2v[0, :]
    a2d = a2v[1, :]

    def group_body(g, carry):
        @plsc.parallel_loop(0, IGRP * 128 // L, unroll=4)
        def vec_body(v):
            j = v >> 3
            k = v & 7
            sv = sidx[g * IGRP + j, pl.ds(k * L, L)]
            dv = didx[g * IGRP + j, pl.ds(k * L, L)]
            hs = plsc.load_gather(h2t, [sv])
            hd = plsc.load_gather(h2t, [dv])
            al = a2s * hs + a2d * hd
            ea = jnp.exp(jnp.maximum(al, 0.2 * al))
            gg = j * 128 + k * L + lane
            plsc.store_scatter(val, [gg, zero16], ea * hs)
            plsc.store_scatter(val, [gg, one16], ea)
        for j in range(IGRP):
            pltpu.sync_copy(val.at[pl.ds(j * 128, 128)],
                            acc2.at[didx.at[g * IGRP + j]], add=True)
        return carry

    lax.fori_loop(0, NGRP, group_body, 0)
    plsc.subcore_barrier()
    pltpu.sync_copy(acc2.at[pl.ds(s * RPN, RPN)],
                    out2.at[c, pl.ds(s * RPN, RPN)])


_s2 = pl.kernel(
    _s2_body,
    out_type=jax.ShapeDtypeStruct((NC, NP, 2 * H), jnp.float32),
    mesh=_MESH,
    compiler_params=_SC_PARAMS,
    scratch_types=[
        pltpu.VMEM((RPT, 128), jnp.int32),
        pltpu.VMEM((RPT, 128), jnp.int32),
        pltpu.VMEM((NP,), jnp.float32),
        pltpu.VMEM((GEDGE, 2 * H), jnp.float32),
        pltpu.VMEM((2, L), jnp.float32),
        pltpu.VMEM_SHARED((NP, 2 * H), jnp.float32),
        pltpu.SemaphoreType.DMA,
    ],
)


def _t3_body(o2_ref, h2_ref, sc_ref, out_ref):
    a2s = sc_ref[0, 0]
    a2d = sc_ref[0, 1]
    b2v = sc_ref[0, 2]
    h2 = h2_ref[...]
    sa = (a2s + a2d) * h2
    sea = jnp.exp(jnp.maximum(sa, 0.2 * sa))
    num = o2_ref[0][:, 0:1] + o2_ref[1][:, 0:1] + sea * h2
    den = o2_ref[0][:, 1:2] + o2_ref[1][:, 1:2] + sea
    node = num / (den + 1e-16)
    valid = lax.broadcasted_iota(jnp.int32, (NP, 1), 0) < N
    node = jnp.where(valid, node, 0.0)
    out_ref[...] = (jnp.sum(node) / N + b2v).reshape(1, 1)


_t3 = pl.pallas_call(
    _t3_body,
    out_shape=jax.ShapeDtypeStruct((1, 1), jnp.float32),
)


def kernel(x, edge_index, W1, a_src1, a_dst1, b1, W2, a_src2, a_dst2, b2):
    x = x.astype(jnp.float32)
    src = edge_index[0].astype(jnp.int32)
    dst = edge_index[1].astype(jnp.int32)
    srcr = jnp.pad(src, (0, EP - E), constant_values=N).reshape(EROWS, 128)
    dstr = jnp.pad(dst, (0, EP - E), constant_values=N).reshape(EROWS, 128)
    xp = jnp.pad(x, ((0, NP - N), (0, 0)))

    eye = jnp.eye(H, dtype=jnp.float32)
    a_mat = jnp.concatenate(
        [(eye[:, None, :] * a_src1[:, :, None]).reshape(HD, H),
         (eye[:, None, :] * a_dst1[:, :, None]).reshape(HD, H)], axis=1)

    ntab, adr = _t1(xp, W1, a_mat)

    z128 = jnp.zeros((NP, NT), jnp.float32)
    outp = _s1(srcr, dstr, ntab, adr, z128)

    h2 = _t2(outp, ntab, b1, W2)

    a2v = jnp.stack([jnp.full((L,), a_src2[0, 0], jnp.float32),
                     jnp.full((L,), a_dst2[0, 0], jnp.float32)])
    z2 = jnp.zeros((NP, 2 * H), jnp.float32)
    out2 = _s2(srcr, dstr, h2[:, 0], z2, a2v)

    sc3 = jnp.stack([a_src2[0, 0], a_dst2[0, 0], b2[0],
                     jnp.float32(0.0)]).reshape(1, 4)
    return _t3(out2, h2, sc3)
